# baseline (speedup 1.0000x reference)
import functools

import jax
import jax.numpy as jnp
from jax import lax
from jax.experimental import pallas as pl
from jax.experimental.pallas import tpu as pltpu
from jax.experimental.pallas import tpu_sc as plsc

D_MODEL = 64
NUM_CORES = 2
NUM_SUBCORES = 16
NUM_WORKERS = NUM_CORES * NUM_SUBCORES
CHUNK = 128
NBUF = 2


@functools.lru_cache(maxsize=None)
def _make_lookup(B: int):
    assert B % (NUM_WORKERS * CHUNK) == 0
    b_per_w = B // NUM_WORKERS
    nchunks = b_per_w // CHUNK
    assert nchunks % NBUF == 0
    mesh = plsc.VectorSubcoreMesh(core_axis_name="c", subcore_axis_name="s")

    @functools.partial(
        pl.kernel,
        mesh=mesh,
        out_type=jax.ShapeDtypeStruct((B, D_MODEL), jnp.float32),
        compiler_params=pltpu.CompilerParams(use_tc_tiling_on_sc=False),
        scratch_types=[
            pltpu.VMEM((nchunks, CHUNK), jnp.int32),
            pltpu.VMEM((CHUNK, D_MODEL), jnp.float32),
            pltpu.VMEM((CHUNK, D_MODEL), jnp.float32),
            pltpu.SemaphoreType.DMA,
            pltpu.SemaphoreType.DMA,
        ],
    )
    def lookup(idx_hbm, table_hbm, out_hbm, idx_v, rows0, rows1, sem0, sem1):
        wid = lax.axis_index("s") * NUM_CORES + lax.axis_index("c")
        base = wid * b_per_w
        rows = (rows0, rows1)
        sems = (sem0, sem1)

        pltpu.sync_copy(idx_hbm.at[wid], idx_v)

        for b in range(NBUF):
            pltpu.async_copy(table_hbm.at[idx_v.at[b]], rows[b], sems[b])

        def body(i, carry):
            j0 = i * NBUF
            for b in range(NBUF):
                j = j0 + b
                pltpu.make_async_copy(
                    table_hbm.at[idx_v.at[j]], rows[b], sems[b]
                ).wait()
                pltpu.sync_copy(
                    rows[b], out_hbm.at[pl.ds(base + j * CHUNK, CHUNK)]
                )
                nxt = j + NBUF

                @pl.when(nxt < nchunks)
                def _():
                    pltpu.async_copy(
                        table_hbm.at[idx_v.at[nxt]], rows[b], sems[b]
                    )

            return carry

        lax.fori_loop(0, nchunks // NBUF, body, 0)

    return lookup


def kernel(token_ids, weight):
    s0, s1 = token_ids.shape
    B = s0 * s1
    idx = token_ids.reshape(-1).astype(jnp.int32)
    idx = idx.reshape(NUM_WORKERS, B // (NUM_WORKERS * CHUNK), CHUNK)
    out = _make_lookup(B)(idx, weight)
    return out.reshape(s0, s1, D_MODEL)


# speedup vs baseline: 1.0234x; 1.0234x over previous
import functools

import jax
import jax.numpy as jnp
from jax import lax
from jax.experimental import pallas as pl
from jax.experimental.pallas import tpu as pltpu
from jax.experimental.pallas import tpu_sc as plsc

D_MODEL = 64
NUM_CORES = 2
NUM_SUBCORES = 16
NUM_WORKERS = NUM_CORES * NUM_SUBCORES
CHUNK = 512
NBUF = 2


@functools.lru_cache(maxsize=None)
def _make_lookup(B: int):
    assert B % (NUM_WORKERS * CHUNK) == 0
    b_per_w = B // NUM_WORKERS
    nchunks = b_per_w // CHUNK
    assert nchunks % NBUF == 0
    mesh = plsc.VectorSubcoreMesh(core_axis_name="c", subcore_axis_name="s")

    @functools.partial(
        pl.kernel,
        mesh=mesh,
        out_type=jax.ShapeDtypeStruct((B, D_MODEL), jnp.float32),
        compiler_params=pltpu.CompilerParams(use_tc_tiling_on_sc=False),
        scratch_types=[
            pltpu.VMEM((nchunks, CHUNK), jnp.int32),
            pltpu.VMEM((CHUNK, D_MODEL), jnp.float32),
            pltpu.VMEM((CHUNK, D_MODEL), jnp.float32),
            pltpu.SemaphoreType.DMA,
            pltpu.SemaphoreType.DMA,
        ],
    )
    def lookup(idx_hbm, table_hbm, out_hbm, idx_v, rows0, rows1, sem0, sem1):
        wid = lax.axis_index("s") * NUM_CORES + lax.axis_index("c")
        base = wid * b_per_w
        rows = (rows0, rows1)
        sems = (sem0, sem1)

        pltpu.sync_copy(idx_hbm.at[wid], idx_v)

        for b in range(NBUF):
            pltpu.async_copy(table_hbm.at[idx_v.at[b]], rows[b], sems[b])

        def body(i, carry):
            j0 = i * NBUF
            for b in range(NBUF):
                j = j0 + b
                pltpu.make_async_copy(
                    table_hbm.at[idx_v.at[j]], rows[b], sems[b]
                ).wait()
                pltpu.sync_copy(
                    rows[b], out_hbm.at[pl.ds(base + j * CHUNK, CHUNK)]
                )
                nxt = j + NBUF

                @pl.when(nxt < nchunks)
                def _():
                    pltpu.async_copy(
                        table_hbm.at[idx_v.at[nxt]], rows[b], sems[b]
                    )

            return carry

        lax.fori_loop(0, nchunks // NBUF, body, 0)

    return lookup


def kernel(token_ids, weight):
    s0, s1 = token_ids.shape
    B = s0 * s1
    idx = token_ids.reshape(-1).astype(jnp.int32)
    idx = idx.reshape(NUM_WORKERS, B // (NUM_WORKERS * CHUNK), CHUNK)
    out = _make_lookup(B)(idx, weight)
    return out.reshape(s0, s1, D_MODEL)


# speedup vs baseline: 1.3593x; 1.3282x over previous
import functools

import jax
import jax.numpy as jnp
from jax import lax
from jax.experimental import pallas as pl
from jax.experimental.pallas import tpu as pltpu
from jax.experimental.pallas import tpu_sc as plsc

D_MODEL = 64
D_PAD = 128
NUM_CORES = 2
NUM_SUBCORES = 16
NUM_WORKERS = NUM_CORES * NUM_SUBCORES
CHUNK = 512
NBUF = 2


@functools.lru_cache(maxsize=None)
def _make_lookup(B: int):
    assert B % (NUM_WORKERS * CHUNK) == 0
    b_per_w = B // NUM_WORKERS
    nchunks = b_per_w // CHUNK
    assert nchunks % NBUF == 0
    mesh = plsc.VectorSubcoreMesh(core_axis_name="c", subcore_axis_name="s")

    @functools.partial(
        pl.kernel,
        mesh=mesh,
        out_type=jax.ShapeDtypeStruct((B, D_PAD), jnp.float32),
        compiler_params=pltpu.CompilerParams(use_tc_tiling_on_sc=False),
        scratch_types=[
            pltpu.VMEM((b_per_w,), jnp.int32),
            pltpu.VMEM((CHUNK, D_MODEL), jnp.float32),
            pltpu.VMEM((CHUNK, D_MODEL), jnp.float32),
            pltpu.SemaphoreType.DMA,
            pltpu.SemaphoreType.DMA,
        ],
    )
    def lookup(idx_hbm, table_hbm, out_hbm, idx_v, rows0, rows1, sem0, sem1):
        wid = lax.axis_index("s") * NUM_CORES + lax.axis_index("c")
        base = wid * b_per_w
        rows = (rows0, rows1)
        sems = (sem0, sem1)

        pltpu.sync_copy(idx_hbm.at[wid], idx_v)

        for b in range(NBUF):
            pltpu.async_copy(
                table_hbm.at[idx_v.at[pl.ds(b * CHUNK, CHUNK)]],
                rows[b],
                sems[b],
            )

        def body(i, carry):
            j0 = i * NBUF
            for b in range(NBUF):
                j = j0 + b
                pltpu.make_async_copy(
                    table_hbm.at[idx_v.at[pl.ds(j * CHUNK, CHUNK)]],
                    rows[b],
                    sems[b],
                ).wait()
                pltpu.sync_copy(
                    rows[b],
                    out_hbm.at[pl.ds(base + j * CHUNK, CHUNK), pl.ds(0, D_MODEL)],
                )
                nxt = j + NBUF

                @pl.when(nxt < nchunks)
                def _():
                    pltpu.async_copy(
                        table_hbm.at[idx_v.at[pl.ds(nxt * CHUNK, CHUNK)]],
                        rows[b],
                        sems[b],
                    )

            return carry

        lax.fori_loop(0, nchunks // NBUF, body, 0)

    return lookup


def kernel(token_ids, weight):
    s0, s1 = token_ids.shape
    B = s0 * s1
    idx = token_ids.astype(jnp.int32).reshape(NUM_WORKERS, B // NUM_WORKERS)
    out = _make_lookup(B)(idx, weight)
    return out[:, :D_MODEL].reshape(s0, s1, D_MODEL)
